# baseline (speedup 1.0000x reference)
import functools
import math

import jax
import jax.numpy as jnp
from jax import lax
from jax.experimental import pallas as pl
from jax.experimental.pallas import tpu as pltpu
from jax.experimental.pallas import tpu_sc as plsc

VOCAB = 1000000
D = 64
SCALE = math.sqrt(D)
NC, NS = 2, 16
NW = NC * NS
CH = 512
NBUF = 3


def _emb_body(B, BPW, NCHUNK,
              x_hbm, table_hbm, out_hbm,
              idx_all, r0, r1, r2, g0, g1, g2, o0, o1, o2):
    rows = (r0, r1, r2)
    gsem = (g0, g1, g2)
    osem = (o0, o1, o2)
    wid = lax.axis_index("s") * NC + lax.axis_index("c")
    base = wid * BPW

    pltpu.sync_copy(x_hbm.at[pl.ds(base, BPW)], idx_all)

    def start_gather(c, b):
        pltpu.async_copy(table_hbm.at[idx_all.at[pl.ds(c * CH, CH)]],
                         rows[b], gsem[b])

    def wait_gather(c, b):
        pltpu.make_async_copy(table_hbm.at[idx_all.at[pl.ds(c * CH, CH)]],
                              rows[b], gsem[b]).wait()

    def scale(b):
        r = rows[b]

        @plsc.parallel_loop(0, CH, unroll=4)
        def _(i):
            for col in range(D // 16):
                sl = (i, pl.ds(col * 16, 16))
                r[sl] = r[sl] * SCALE

    def start_out(c, b):
        pltpu.async_copy(rows[b], out_hbm.at[pl.ds(base + c * CH, CH)],
                         osem[b])

    def wait_out(c, b):
        pltpu.make_async_copy(rows[b], out_hbm.at[pl.ds(base + c * CH, CH)],
                              osem[b]).wait()

    def drain_body(c, b):
        wait_gather(c, b)
        scale(b)
        start_out(c, b)

    def uniform_body(c, b):
        bb = (b + 2) % NBUF
        wait_out(c - 1, bb)
        start_gather(c + 2, bb)
        drain_body(c, b)

    start_gather(0, 0)
    start_gather(1, 1)
    drain_body(0, 0)
    start_gather(2, 2)
    uniform_body(1, 1)

    n_uniform = NCHUNK - 5
    assert n_uniform % NBUF == 0

    def outer(t, _):
        c0 = 2 + t * NBUF
        for j in range(NBUF):
            uniform_body(c0 + j, (2 + j) % NBUF)
        return 0

    lax.fori_loop(0, n_uniform // NBUF, outer, 0)

    uniform_body(NCHUNK - 3, (NCHUNK - 3) % NBUF)
    drain_body(NCHUNK - 2, (NCHUNK - 2) % NBUF)
    drain_body(NCHUNK - 1, (NCHUNK - 1) % NBUF)
    wait_out(NCHUNK - 3, (NCHUNK - 3) % NBUF)
    wait_out(NCHUNK - 2, (NCHUNK - 2) % NBUF)
    wait_out(NCHUNK - 1, (NCHUNK - 1) % NBUF)


@functools.partial(jax.jit, static_argnames=("B",))
def _emb(xf, table, B):
    BPW = B // NW
    NCHUNK = BPW // CH
    body = functools.partial(_emb_body, B, BPW, NCHUNK)
    run = pl.kernel(
        body,
        out_type=jax.ShapeDtypeStruct((B, D), jnp.float32),
        mesh=plsc.VectorSubcoreMesh(core_axis_name="c", subcore_axis_name="s",
                                    num_cores=NC, num_subcores=NS),
        compiler_params=pltpu.CompilerParams(use_tc_tiling_on_sc=False),
        scratch_types=[
            pltpu.VMEM((BPW,), jnp.int32),
            pltpu.VMEM((CH, D), jnp.float32),
            pltpu.VMEM((CH, D), jnp.float32),
            pltpu.VMEM((CH, D), jnp.float32),
            pltpu.SemaphoreType.DMA,
            pltpu.SemaphoreType.DMA,
            pltpu.SemaphoreType.DMA,
            pltpu.SemaphoreType.DMA,
            pltpu.SemaphoreType.DMA,
            pltpu.SemaphoreType.DMA,
        ],
    )
    return run(xf, table)


def kernel(x, table):
    B = x.shape[0] * x.shape[1]
    xf = x.reshape(B).astype(jnp.int32)
    out = _emb(xf, table, B)
    return out.reshape(x.shape[0], x.shape[1], D)
